# Initial kernel scaffold; baseline (speedup 1.0000x reference)
import jax
import jax.numpy as jnp
from jax.experimental import pallas as pl


def kernel(node_feats, edge_feats, tab_feats, edge_index, graph_ids, gc_Wpn, gc_bpn, gc_Wpe1, gc_bpe1, gc_Wpe2, gc_bpe2, gc_Wet, gc_bet, gc_Wih, gc_Whh, gc_bih, gc_bhh, gl_Wpe, gl_bpe, gl_Wpn, gl_bpn, gl_Wih, gl_Whh, gl_bih, gl_bhh, ro_Wz, ro_bz, ro_Wp, ro_bp, ro_Wih, ro_Whh, ro_bih, ro_bhh, d0_W, d0_b, d1_W, d1_b, pr_W, pr_b):
    raise NotImplementedError("write your pallas kernel here")


# speedup vs baseline: 3.8817x; 3.8817x over previous
import functools
import jax
import jax.numpy as jnp
from jax import lax
from jax.experimental import pallas as pl
from jax.experimental.pallas import tpu as pltpu
from jax.experimental.pallas import tpu_sc as plsc

N, E, B = 10000, 320000, 64
DN, DE, DT, G = 128, 16, 100, 200
GP = 208
NFEAT = GP // 16

NC, NS = 2, 16
HP = 112
NH = HP // 16
EPT = E // NS
EPW = E // (NC * NS)
KC = 80
NCHUNK = EPT // KC
NCHUNKW = EPW // KC
NGRP = KC // 16
RPT = 624
RZ = 48
NTAIL = N - NS * RPT

f32 = jnp.float32


def _lrelu(x):
    return jnp.where(x >= 0, x, 0.01 * x)


def _elu(x):
    return jnp.where(x > 0, x, jnp.exp(jnp.minimum(x, 0.0)) - 1.0)


def _hiprec(f):
    @functools.wraps(f)
    def g(*a):
        with jax.default_matmul_precision("highest"):
            return f(*a)
    return g


@_hiprec
def _prep_nodes_body(nf_ref, wpn_ref, bpn_ref, wpe1n_ref, w2u_ref, bpe2_ref,
                     hv_ref, pn_ref, u_ref):
    nf = nf_ref[...]
    hv = _lrelu(nf @ wpn_ref[...] + bpn_ref[...])
    hv_ref[...] = hv
    pn_ref[...] = nf @ wpe1n_ref[...]
    u_ref[...] = hv @ w2u_ref[...] + bpe2_ref[...]


def _prep_nodes(nf, wpn, bpn, wpe1n, w2u, bpe2):
    blk = 2000
    grid = N // blk
    return pl.pallas_call(
        _prep_nodes_body,
        grid=(grid,),
        in_specs=[
            pl.BlockSpec((blk, DN), lambda i: (i, 0)),
            pl.BlockSpec((DN, G), lambda i: (0, 0)),
            pl.BlockSpec((1, G), lambda i: (0, 0)),
            pl.BlockSpec((DN, GP), lambda i: (0, 0)),
            pl.BlockSpec((G, 1), lambda i: (0, 0)),
            pl.BlockSpec((1, 1), lambda i: (0, 0)),
        ],
        out_specs=[
            pl.BlockSpec((blk, G), lambda i: (i, 0)),
            pl.BlockSpec((blk, GP), lambda i: (i, 0)),
            pl.BlockSpec((blk, 1), lambda i: (i, 0)),
        ],
        out_shape=[
            jax.ShapeDtypeStruct((N, G), f32),
            jax.ShapeDtypeStruct((N, GP), f32),
            jax.ShapeDtypeStruct((N, 1), f32),
        ],
    )(nf, wpn, bpn, wpe1n, w2u, bpe2)


@_hiprec
def _edge_proj_body(ef_ref, w_ref, b_ref, pe_ref):
    pe_ref[...] = ef_ref[...] @ w_ref[...] + b_ref[...]


def _edge_proj(ef, w, b):
    blk = 8000
    grid = E // blk
    return pl.pallas_call(
        _edge_proj_body,
        grid=(grid,),
        in_specs=[
            pl.BlockSpec((blk, DE), lambda i: (i, 0)),
            pl.BlockSpec((DE, GP), lambda i: (0, 0)),
            pl.BlockSpec((1, GP), lambda i: (0, 0)),
        ],
        out_specs=pl.BlockSpec((blk, GP), lambda i: (i, 0)),
        out_shape=jax.ShapeDtypeStruct((E, GP), f32),
    )(ef, w, b)


@_hiprec
def _post_round_body(c2_ref, h_ref, wt_ref, bt_ref, wih_ref, whh_ref,
                     bih_ref, bhh_ref, wu_ref, bu_ref, wv_ref,
                     hlo_ref, hhi_ref, u_ref, v_ref):
    lo = c2_ref[0]
    hi = c2_ref[1]
    s = hi[:, G - HP:G - HP + 1]
    mask = s > 0
    sden = jnp.where(mask, s, 1.0)
    C = jnp.concatenate([lo, hi[:, :G - HP]], axis=1)
    Cn = jnp.where(mask, C / sden, 0.0)
    c = Cn @ wt_ref[...] + jnp.where(mask, 1.0, 0.0) * bt_ref[...]
    x = _elu(c)
    h = h_ref[...]
    gi = x @ wih_ref[...] + bih_ref[...]
    gh = h @ whh_ref[...] + bhh_ref[...]
    r = jax.nn.sigmoid(gi[:, :G] + gh[:, :G])
    z = jax.nn.sigmoid(gi[:, G:2 * G] + gh[:, G:2 * G])
    nn_ = jnp.tanh(gi[:, 2 * G:] + r * gh[:, 2 * G:])
    hn = jnp.maximum((1.0 - z) * nn_ + z * h, 0.0)
    hlo_ref[...] = hn[:, :HP]
    hhi_ref[...] = jnp.concatenate(
        [hn[:, HP:], jnp.zeros((hn.shape[0], 2 * HP - G), f32)], axis=1)
    u_ref[...] = hn @ wu_ref[...] + bu_ref[...]
    v_ref[...] = hn @ wv_ref[...]


def _post_round(c2, h, wt, bt, wih, whh, bih, bhh, wu, bu, wv):
    blk = 2000
    grid = N // blk
    return pl.pallas_call(
        _post_round_body,
        grid=(grid,),
        in_specs=[
            pl.BlockSpec((NC, blk, HP), lambda i: (0, i, 0)),
            pl.BlockSpec((blk, G), lambda i: (i, 0)),
            pl.BlockSpec((G, G), lambda i: (0, 0)),
            pl.BlockSpec((1, G), lambda i: (0, 0)),
            pl.BlockSpec((G, 3 * G), lambda i: (0, 0)),
            pl.BlockSpec((G, 3 * G), lambda i: (0, 0)),
            pl.BlockSpec((1, 3 * G), lambda i: (0, 0)),
            pl.BlockSpec((1, 3 * G), lambda i: (0, 0)),
            pl.BlockSpec((G, 1), lambda i: (0, 0)),
            pl.BlockSpec((1, 1), lambda i: (0, 0)),
            pl.BlockSpec((G, 1), lambda i: (0, 0)),
        ],
        out_specs=[
            pl.BlockSpec((blk, HP), lambda i: (i, 0)),
            pl.BlockSpec((blk, HP), lambda i: (i, 0)),
            pl.BlockSpec((blk, 1), lambda i: (i, 0)),
            pl.BlockSpec((blk, 1), lambda i: (i, 0)),
        ],
        out_shape=[
            jax.ShapeDtypeStruct((N, HP), f32),
            jax.ShapeDtypeStruct((N, HP), f32),
            jax.ShapeDtypeStruct((N, 1), f32),
            jax.ShapeDtypeStruct((N, 1), f32),
        ],
    )(c2, h, wt, bt, wih, whh, bih, bhh, wu, bu, wv)


def _edge_exp_body(lg_ref, e_ref):
    e_ref[...] = jnp.exp(lg_ref[...])


def _edge_exp(lg):
    e = pl.pallas_call(
        _edge_exp_body,
        out_shape=jax.ShapeDtypeStruct((E // 128, 128), f32),
    )(lg.reshape(E // 128, 128))
    return e.reshape(E)


def _readout_body(h_ref, gid_ref, tab_ref,
                  wz0t_ref, wz0v_ref, bz0_ref, wp0_ref, bp0_ref,
                  wih0_ref, whh0_ref, bih0_ref, bhh0_ref,
                  wz1t_ref, wz1v_ref, bz1_ref, wp1_ref, bp1_ref,
                  wih1_ref, whh1_ref, bih1_ref, bhh1_ref,
                  d0g_ref, d0t_ref, d0b_ref, d1_ref, d1b_ref,
                  pr_ref, prb_ref, out_ref):
    h = h_ref[...]
    gid = gid_ref[...]
    M = (jnp.broadcast_to(gid, (B, N)) ==
         lax.broadcasted_iota(jnp.int32, (B, N), 0)).astype(f32)
    g = M @ h

    def round_(g, wzt, wzv, bz, wp, bp, wih, whh, bih, bhh):
        ztg = jnp.maximum(g, 0.0) @ wzt + bz
        zv = lax.dot_general(wzv, h, (((0,), (1,)), ((), ())))
        ztn = lax.dot_general(ztg, M, (((0,), (0,)), ((), ())))
        e = jnp.exp(_lrelu(ztn + zv))
        Me = M * e
        sb = jnp.sum(Me, axis=1, keepdims=True)
        hp = h @ wp + bp
        Cg = Me @ hp
        mask = sb > 0
        grp_ = jnp.where(mask, Cg / jnp.where(mask, sb, 1.0), 0.0)
        x = _elu(grp_)
        gi = x @ wih + bih
        gh = g @ whh + bhh
        r = jax.nn.sigmoid(gi[:, :G] + gh[:, :G])
        z = jax.nn.sigmoid(gi[:, G:2 * G] + gh[:, G:2 * G])
        nn_ = jnp.tanh(gi[:, 2 * G:] + r * gh[:, 2 * G:])
        return jnp.maximum((1.0 - z) * nn_ + z * g, 0.0)

    g = round_(g, wz0t_ref[...], wz0v_ref[...], bz0_ref[...], wp0_ref[...],
               bp0_ref[...], wih0_ref[...], whh0_ref[...], bih0_ref[...],
               bhh0_ref[...])
    g = round_(g, wz1t_ref[...], wz1v_ref[...], bz1_ref[...], wp1_ref[...],
               bp1_ref[...], wih1_ref[...], whh1_ref[...], bih1_ref[...],
               bhh1_ref[...])

    x = jnp.maximum(g @ d0g_ref[...] + tab_ref[...] @ d0t_ref[...]
                    + d0b_ref[...], 0.0)
    x = jnp.maximum(x @ d1_ref[...] + d1b_ref[...], 0.0)
    out_ref[...] = x @ pr_ref[...] + prb_ref[...]


def _readout(h, gid2d, tab, *ws):
    return pl.pallas_call(
        _readout_body,
        out_shape=jax.ShapeDtypeStruct((B, 1), f32),
    )(h, gid2d, tab, *ws)


@functools.cache
def _mesh():
    return plsc.VectorSubcoreMesh(
        core_axis_name="c", subcore_axis_name="s",
        num_cores=NC, num_subcores=NS)


_SC_PARAMS = pltpu.CompilerParams(
    use_tc_tiling_on_sc=False, needs_layout_passes=False)


def _zero_shared(Cs, buf_v, sid):
    z16 = jnp.zeros((16,), f32)

    def zrow(r, _):
        def zcol(j, _):
            buf_v[r, pl.ds(j * 16, 16)] = z16
            return 0
        return lax.fori_loop(0, NH, zcol, 0)

    lax.fori_loop(0, RZ, zrow, 0)

    def zcopy(k, _):
        pltpu.sync_copy(buf_v, Cs.at[pl.ds(sid * RPT + k * RZ, RZ)])
        return 0

    lax.fori_loop(0, RPT // RZ, zcopy, 0)

    @pl.when(sid == NS - 1)
    def _():
        pltpu.sync_copy(buf_v.at[pl.ds(0, NTAIL)],
                        Cs.at[pl.ds(NS * RPT, NTAIL)])


def _write_shared(Cs, buf_v, out_hbm, cid, sid):
    def wcopy(k, _):
        row0 = sid * RPT + k * RZ
        pltpu.sync_copy(Cs.at[pl.ds(row0, RZ)], buf_v)
        pltpu.sync_copy(buf_v, out_hbm.at[cid, pl.ds(row0, RZ)])
        return 0

    lax.fori_loop(0, RPT // RZ, wcopy, 0)

    @pl.when(sid == NS - 1)
    def _():
        pltpu.sync_copy(Cs.at[pl.ds(NS * RPT, NTAIL)],
                        buf_v.at[pl.ds(0, NTAIL)])
        pltpu.sync_copy(buf_v.at[pl.ds(0, NTAIL)],
                        out_hbm.at[cid, pl.ds(NS * RPT, NTAIL)])


def _ebcast(e_v, i):
    return plsc.load_gather(e_v, [jnp.full((16,), i, jnp.int32)])


def _sc_ctx_logits_body(src_hbm, dst_hbm, pn_hbm, pelo_hbm, pehi_hbm, u_hbm,
                        w2b_hbm, lg_hbm,
                        u_v, w2b_v, src_v, src2_v, dst_v, rlo_v, rhi_v,
                        plo_v, phi_v, dotp_v, lg_v, sem):
    cid = lax.axis_index("c")
    sid = lax.axis_index("s")
    wid = cid * NS + sid
    pltpu.sync_copy(u_hbm, u_v)
    pltpu.sync_copy(w2b_hbm, w2b_v)

    base0 = wid * EPW

    def chunk(t, _):
        base = base0 + t * KC
        pltpu.sync_copy(src_hbm.at[pl.ds(base, KC)], src_v)
        pltpu.sync_copy(dst_hbm.at[pl.ds(base, KC)], dst_v)

        def g0(gi, _):
            src2_v[pl.ds(gi * 16, 16)] = src_v[pl.ds(gi * 16, 16)] + N
            return 0

        lax.fori_loop(0, NGRP, g0, 0)
        pltpu.async_copy(pn_hbm.at[src_v], rlo_v, sem).wait()
        pltpu.async_copy(pn_hbm.at[src2_v], rhi_v, sem).wait()
        pltpu.sync_copy(pelo_hbm.at[pl.ds(base, KC)], plo_v)
        pltpu.sync_copy(pehi_hbm.at[pl.ds(base, KC)], phi_v)

        def edge(i, _):
            def flo(j, acc):
                q = rlo_v[i, pl.ds(j * 16, 16)] + plo_v[i, pl.ds(j * 16, 16)]
                he = jnp.where(q >= 0, q, 0.01 * q)
                return acc + he * w2b_v[pl.ds(j * 16, 16)]

            def fhi(j, acc):
                q = rhi_v[i, pl.ds(j * 16, 16)] + phi_v[i, pl.ds(j * 16, 16)]
                he = jnp.where(q >= 0, q, 0.01 * q)
                return acc + he * w2b_v[pl.ds((NH + j) * 16, 16)]

            acc = lax.fori_loop(0, NH, flo, jnp.zeros((16,), f32))
            acc = lax.fori_loop(0, NH - 1, fhi, acc)
            dotp_v[i, pl.ds(0, 16)] = acc
            return 0

        lax.fori_loop(0, KC, edge, 0)

        def grp(gi, _):
            eids = lax.iota(jnp.int32, 16) + gi * 16
            dstv = dst_v[pl.ds(gi * 16, 16)]
            uv = plsc.load_gather(u_v, [dstv])

            def dj(j, dv):
                jv = jnp.full((16,), j, jnp.int32)
                return dv + plsc.load_gather(dotp_v, [eids, jv])

            dv = lax.fori_loop(0, 16, dj, jnp.zeros((16,), f32))
            lg = uv + dv
            lg_v[pl.ds(gi * 16, 16)] = jnp.where(lg >= 0, lg, 0.01 * lg)
            return 0

        lax.fori_loop(0, NGRP, grp, 0)
        pltpu.sync_copy(lg_v, lg_hbm.at[pl.ds(base, KC)])
        return 0

    lax.fori_loop(0, NCHUNKW, chunk, 0)


def _sc_lay_logits_body(src_hbm, dst_hbm, u_hbm, v_hbm, lg_hbm,
                        u_v, v_v, src_v, dst_v, lg_v):
    cid = lax.axis_index("c")
    sid = lax.axis_index("s")
    wid = cid * NS + sid
    pltpu.sync_copy(u_hbm, u_v)
    pltpu.sync_copy(v_hbm, v_v)

    base0 = wid * EPW

    def chunk(t, _):
        base = base0 + t * KC
        pltpu.sync_copy(src_hbm.at[pl.ds(base, KC)], src_v)
        pltpu.sync_copy(dst_hbm.at[pl.ds(base, KC)], dst_v)

        def grp(gi, _):
            srcv = src_v[pl.ds(gi * 16, 16)]
            dstv = dst_v[pl.ds(gi * 16, 16)]
            uv = plsc.load_gather(u_v, [dstv])
            vv = plsc.load_gather(v_v, [srcv])
            lg = uv + vv
            lg_v[pl.ds(gi * 16, 16)] = jnp.where(lg >= 0, lg, 0.01 * lg)
            return 0

        lax.fori_loop(0, NGRP, grp, 0)
        pltpu.sync_copy(lg_v, lg_hbm.at[pl.ds(base, KC)])
        return 0

    lax.fori_loop(0, NCHUNKW, chunk, 0)


def _sc_scatter_body(src_hbm, dst_hbm, tab_hbm, e_hbm, out_hbm,
                     Cs, src_v, src2_v, dst_v, rows_v, e_v, buf_v, sem):
    cid = lax.axis_index("c")
    sid = lax.axis_index("s")
    _zero_shared(Cs, buf_v, sid)
    plsc.subcore_barrier()

    base0 = sid * EPT
    z16 = jnp.zeros((16,), f32)
    lane8 = lax.iota(jnp.int32, 16) == 8
    rowoff = cid * N

    def chunk(t, _):
        base = base0 + t * KC
        pltpu.sync_copy(src_hbm.at[pl.ds(base, KC)], src_v)
        pltpu.sync_copy(dst_hbm.at[pl.ds(base, KC)], dst_v)
        pltpu.sync_copy(e_hbm.at[pl.ds(base, KC)], e_v)

        def g0(gi, _):
            src2_v[pl.ds(gi * 16, 16)] = src_v[pl.ds(gi * 16, 16)] + rowoff
            return 0

        lax.fori_loop(0, NGRP, g0, 0)
        pltpu.async_copy(tab_hbm.at[src2_v], rows_v, sem).wait()

        def scale(i, _):
            esv = _ebcast(e_v, i)
            eblend = jnp.where(jnp.logical_and(lane8, cid == 1), esv, z16)

            def sj(j, _):
                rows_v[i, pl.ds(j * 16, 16)] = (
                    rows_v[i, pl.ds(j * 16, 16)] * esv)
                return 0

            lax.fori_loop(0, NH - 2, sj, 0)
            rows_v[i, pl.ds((NH - 2) * 16, 16)] = (
                rows_v[i, pl.ds((NH - 2) * 16, 16)] * esv + eblend)
            rows_v[i, pl.ds((NH - 1) * 16, 16)] = (
                rows_v[i, pl.ds((NH - 1) * 16, 16)] * esv)
            return 0

        lax.fori_loop(0, KC, scale, 0)
        pltpu.sync_copy(rows_v, Cs.at[dst_v], add=True)
        return 0

    lax.fori_loop(0, NCHUNK, chunk, 0)
    plsc.subcore_barrier()
    _write_shared(Cs, buf_v, out_hbm, cid, sid)


def _sc_ctx_scatter_body(src_hbm, dst_hbm, pn_hbm, pelo_hbm, pehi_hbm,
                         e_hbm, out_hbm,
                         Cs, src_v, src2_v, dst_v, rows_v, pe_v, e_v, buf_v,
                         sem):
    cid = lax.axis_index("c")
    sid = lax.axis_index("s")
    _zero_shared(Cs, buf_v, sid)
    plsc.subcore_barrier()

    base0 = sid * EPT
    z16 = jnp.zeros((16,), f32)
    lane8 = lax.iota(jnp.int32, 16) == 8
    rowoff = cid * N

    def chunk(t, _):
        base = base0 + t * KC
        pltpu.sync_copy(src_hbm.at[pl.ds(base, KC)], src_v)
        pltpu.sync_copy(dst_hbm.at[pl.ds(base, KC)], dst_v)
        pltpu.sync_copy(e_hbm.at[pl.ds(base, KC)], e_v)

        def g0(gi, _):
            src2_v[pl.ds(gi * 16, 16)] = src_v[pl.ds(gi * 16, 16)] + rowoff
            return 0

        lax.fori_loop(0, NGRP, g0, 0)
        pltpu.async_copy(pn_hbm.at[src2_v], rows_v, sem).wait()

        @pl.when(cid == 0)
        def _():
            pltpu.sync_copy(pelo_hbm.at[pl.ds(base, KC)], pe_v)

        @pl.when(cid == 1)
        def _():
            pltpu.sync_copy(pehi_hbm.at[pl.ds(base, KC)], pe_v)

        def scale(i, _):
            esv = _ebcast(e_v, i)
            eblend = jnp.where(jnp.logical_and(lane8, cid == 1), esv, z16)

            def sj(j, _):
                q = rows_v[i, pl.ds(j * 16, 16)] + pe_v[i, pl.ds(j * 16, 16)]
                he = jnp.where(q >= 0, q, 0.01 * q)
                rows_v[i, pl.ds(j * 16, 16)] = he * esv
                return 0

            lax.fori_loop(0, NH - 2, sj, 0)
            j = NH - 2
            q = rows_v[i, pl.ds(j * 16, 16)] + pe_v[i, pl.ds(j * 16, 16)]
            he = jnp.where(q >= 0, q, 0.01 * q)
            rows_v[i, pl.ds(j * 16, 16)] = he * esv + eblend
            j = NH - 1
            q = rows_v[i, pl.ds(j * 16, 16)] + pe_v[i, pl.ds(j * 16, 16)]
            he = jnp.where(q >= 0, q, 0.01 * q)
            rows_v[i, pl.ds(j * 16, 16)] = he * esv
            return 0

        lax.fori_loop(0, KC, scale, 0)
        pltpu.sync_copy(rows_v, Cs.at[dst_v], add=True)
        return 0

    lax.fori_loop(0, NCHUNK, chunk, 0)
    plsc.subcore_barrier()
    _write_shared(Cs, buf_v, out_hbm, cid, sid)


@functools.cache
def _sc_ctx_logits_kernel():
    return pl.kernel(
        _sc_ctx_logits_body,
        out_type=jax.ShapeDtypeStruct((E,), f32),
        mesh=_mesh(),
        compiler_params=_SC_PARAMS,
        scratch_types=[
            pltpu.VMEM((N,), f32),
            pltpu.VMEM((GP,), f32),
            pltpu.VMEM((KC,), jnp.int32),
            pltpu.VMEM((KC,), jnp.int32),
            pltpu.VMEM((KC,), jnp.int32),
            pltpu.VMEM((KC, HP), f32),
            pltpu.VMEM((KC, HP), f32),
            pltpu.VMEM((KC, HP), f32),
            pltpu.VMEM((KC, HP), f32),
            pltpu.VMEM((KC, 16), f32),
            pltpu.VMEM((KC,), f32),
            pltpu.SemaphoreType.DMA,
        ],
    )


@functools.cache
def _sc_lay_logits_kernel():
    return pl.kernel(
        _sc_lay_logits_body,
        out_type=jax.ShapeDtypeStruct((E,), f32),
        mesh=_mesh(),
        compiler_params=_SC_PARAMS,
        scratch_types=[
            pltpu.VMEM((N,), f32),
            pltpu.VMEM((N,), f32),
            pltpu.VMEM((KC,), jnp.int32),
            pltpu.VMEM((KC,), jnp.int32),
            pltpu.VMEM((KC,), f32),
        ],
    )


@functools.cache
def _sc_scatter_kernel():
    return pl.kernel(
        _sc_scatter_body,
        out_type=jax.ShapeDtypeStruct((NC, N, HP), f32),
        mesh=_mesh(),
        compiler_params=_SC_PARAMS,
        scratch_types=[
            pltpu.VMEM_SHARED((N, HP), f32),
            pltpu.VMEM((KC,), jnp.int32),
            pltpu.VMEM((KC,), jnp.int32),
            pltpu.VMEM((KC,), jnp.int32),
            pltpu.VMEM((KC, HP), f32),
            pltpu.VMEM((KC,), f32),
            pltpu.VMEM((RZ, HP), f32),
            pltpu.SemaphoreType.DMA,
        ],
    )


@functools.cache
def _sc_ctx_scatter_kernel():
    return pl.kernel(
        _sc_ctx_scatter_body,
        out_type=jax.ShapeDtypeStruct((NC, N, HP), f32),
        mesh=_mesh(),
        compiler_params=_SC_PARAMS,
        scratch_types=[
            pltpu.VMEM_SHARED((N, HP), f32),
            pltpu.VMEM((KC,), jnp.int32),
            pltpu.VMEM((KC,), jnp.int32),
            pltpu.VMEM((KC,), jnp.int32),
            pltpu.VMEM((KC, HP), f32),
            pltpu.VMEM((KC, HP), f32),
            pltpu.VMEM((KC,), f32),
            pltpu.VMEM((RZ, HP), f32),
            pltpu.SemaphoreType.DMA,
        ],
    )


def _sc_ctx(srcv, dstv, pn_stk, pe_lo, pe_hi, u0, w2b):
    lg = _sc_ctx_logits_kernel()(srcv, dstv, pn_stk, pe_lo, pe_hi, u0, w2b)
    e = _edge_exp(lg)
    return _sc_ctx_scatter_kernel()(srcv, dstv, pn_stk, pe_lo, pe_hi, e)


def _sc_layer(srcv, dstv, h_stk, u, v):
    lg = _sc_lay_logits_kernel()(srcv, dstv, u, v)
    e = _edge_exp(lg)
    return _sc_scatter_kernel()(srcv, dstv, h_stk, e)


def kernel(node_feats, edge_feats, tab_feats, edge_index, graph_ids,
           gc_Wpn, gc_bpn, gc_Wpe1, gc_bpe1, gc_Wpe2, gc_bpe2, gc_Wet, gc_bet,
           gc_Wih, gc_Whh, gc_bih, gc_bhh,
           gl_Wpe, gl_bpe, gl_Wpn, gl_bpn, gl_Wih, gl_Whh, gl_bih, gl_bhh,
           ro_Wz, ro_bz, ro_Wp, ro_bp, ro_Wih, ro_Whh, ro_bih, ro_bhh,
           d0_W, d0_b, d1_W, d1_b, pr_W, pr_b):
    src = edge_index[0]
    dst = edge_index[1]

    wpe1n = jnp.pad(gc_Wpe1[:DN], ((0, 0), (0, GP - G)))
    wpe1e = jnp.pad(gc_Wpe1[DN:], ((0, 0), (0, GP - G)))
    bpe1p = jnp.pad(gc_bpe1, (0, GP - G)).reshape(1, GP)
    w2u = gc_Wpe2[:G]
    w2b = jnp.pad(gc_Wpe2[G:, 0], (0, GP - G))
    bpe2 = gc_bpe2.reshape(1, 1)
    row = lambda b: b.reshape(1, -1)

    hv, pn, u0 = _prep_nodes(node_feats, gc_Wpn, row(gc_bpn), wpe1n, w2u, bpe2)
    pe = _edge_proj(edge_feats, wpe1e, bpe1p)
    pad16 = lambda a: jnp.pad(a, ((0, 0), (0, 2 * HP - GP)))
    pn_stk = jnp.concatenate([pn[:, :HP], pad16(pn[:, HP:])], axis=0)
    pe_lo = pe[:, :HP]
    pe_hi = pad16(pe[:, HP:])

    c2 = _sc_ctx(src, dst, pn_stk, pe_lo, pe_hi, u0.reshape(N), w2b)
    hlo, hhi, u, v = _post_round(
        c2, hv, gc_Wet, row(gc_bet), gc_Wih, gc_Whh, row(gc_bih), row(gc_bhh),
        gl_Wpe[0][:G], gl_bpe[0].reshape(1, 1), gl_Wpe[0][G:])

    c2 = _sc_layer(src, dst, jnp.concatenate([hlo, hhi], axis=0),
                   u.reshape(N), v.reshape(N))
    h = jnp.concatenate([hlo, hhi[:, :G - HP]], axis=1)
    hlo, hhi, u, v = _post_round(
        c2, h, gl_Wpn[0], row(gl_bpn[0]), gl_Wih[0], gl_Whh[0],
        row(gl_bih[0]), row(gl_bhh[0]),
        gl_Wpe[1][:G], gl_bpe[1].reshape(1, 1), gl_Wpe[1][G:])

    c2 = _sc_layer(src, dst, jnp.concatenate([hlo, hhi], axis=0),
                   u.reshape(N), v.reshape(N))
    h = jnp.concatenate([hlo, hhi[:, :G - HP]], axis=1)
    hlo, hhi, _, _ = _post_round(
        c2, h, gl_Wpn[1], row(gl_bpn[1]), gl_Wih[1], gl_Whh[1],
        row(gl_bih[1]), row(gl_bhh[1]),
        jnp.zeros((G, 1), f32), jnp.zeros((1, 1), f32), jnp.zeros((G, 1), f32))
    h = jnp.concatenate([hlo, hhi[:, :G - HP]], axis=1)

    ws = (ro_Wz[0][:G], ro_Wz[0][G:], ro_bz[0].reshape(1, 1),
          ro_Wp[0], row(ro_bp[0]),
          ro_Wih[0], ro_Whh[0], row(ro_bih[0]), row(ro_bhh[0]),
          ro_Wz[1][:G], ro_Wz[1][G:], ro_bz[1].reshape(1, 1),
          ro_Wp[1], row(ro_bp[1]),
          ro_Wih[1], ro_Whh[1], row(ro_bih[1]), row(ro_bhh[1]),
          d0_W[:G], d0_W[G:], row(d0_b), d1_W, row(d1_b), pr_W,
          pr_b.reshape(1, 1))
    return _readout(h, graph_ids.reshape(1, N), tab_feats, *ws)


# speedup vs baseline: 5.6850x; 1.4646x over previous
import functools
import jax
import jax.numpy as jnp
from jax import lax
from jax.experimental import pallas as pl
from jax.experimental.pallas import tpu as pltpu
from jax.experimental.pallas import tpu_sc as plsc

N, E, B = 10000, 320000, 64
DN, DE, DT, G = 128, 16, 100, 200
GP = 208
NFEAT = GP // 16

NC, NS = 2, 16
HP = 112
NH = HP // 16
EPT = E // NS
EPW = E // (NC * NS)
KC = 80
NCHUNK = EPT // KC
NCHUNKW = EPW // KC
NGRP = KC // 16
RPT = 624
RZ = 48
NTAIL = N - NS * RPT

f32 = jnp.float32


def _lrelu(x):
    return jnp.where(x >= 0, x, 0.01 * x)


def _elu(x):
    return jnp.where(x > 0, x, jnp.exp(jnp.minimum(x, 0.0)) - 1.0)


def _hiprec(f):
    @functools.wraps(f)
    def g(*a):
        with jax.default_matmul_precision("highest"):
            return f(*a)
    return g


@_hiprec
def _prep_nodes_body(nf_ref, wpn_ref, bpn_ref, wpe1n_ref, w2u_ref, bpe2_ref,
                     hv_ref, pn_ref, u_ref):
    nf = nf_ref[...]
    hv = _lrelu(nf @ wpn_ref[...] + bpn_ref[...])
    hv_ref[...] = hv
    pn_ref[...] = nf @ wpe1n_ref[...]
    u_ref[...] = hv @ w2u_ref[...] + bpe2_ref[...]


def _prep_nodes(nf, wpn, bpn, wpe1n, w2u, bpe2):
    blk = 2000
    grid = N // blk
    return pl.pallas_call(
        _prep_nodes_body,
        grid=(grid,),
        in_specs=[
            pl.BlockSpec((blk, DN), lambda i: (i, 0)),
            pl.BlockSpec((DN, G), lambda i: (0, 0)),
            pl.BlockSpec((1, G), lambda i: (0, 0)),
            pl.BlockSpec((DN, GP), lambda i: (0, 0)),
            pl.BlockSpec((G, 1), lambda i: (0, 0)),
            pl.BlockSpec((1, 1), lambda i: (0, 0)),
        ],
        out_specs=[
            pl.BlockSpec((blk, G), lambda i: (i, 0)),
            pl.BlockSpec((blk, GP), lambda i: (i, 0)),
            pl.BlockSpec((blk, 1), lambda i: (i, 0)),
        ],
        out_shape=[
            jax.ShapeDtypeStruct((N, G), f32),
            jax.ShapeDtypeStruct((N, GP), f32),
            jax.ShapeDtypeStruct((N, 1), f32),
        ],
    )(nf, wpn, bpn, wpe1n, w2u, bpe2)


@_hiprec
def _edge_proj_body(ef_ref, w_ref, b_ref, pe_ref):
    pe_ref[...] = ef_ref[...] @ w_ref[...] + b_ref[...]


def _edge_proj(ef, w, b):
    blk = 8000
    grid = E // blk
    return pl.pallas_call(
        _edge_proj_body,
        grid=(grid,),
        in_specs=[
            pl.BlockSpec((blk, DE), lambda i: (i, 0)),
            pl.BlockSpec((DE, GP), lambda i: (0, 0)),
            pl.BlockSpec((1, GP), lambda i: (0, 0)),
        ],
        out_specs=pl.BlockSpec((blk, GP), lambda i: (i, 0)),
        out_shape=jax.ShapeDtypeStruct((E, GP), f32),
    )(ef, w, b)


@_hiprec
def _post_round_body(c2_ref, h_ref, wt_ref, bt_ref, wih_ref, whh_ref,
                     bih_ref, bhh_ref, wu_ref, bu_ref, wv_ref,
                     hlo_ref, hhi_ref, u_ref, v_ref):
    lo = c2_ref[0]
    hi = c2_ref[1]
    s = hi[:, G - HP:G - HP + 1]
    mask = s > 0
    sden = jnp.where(mask, s, 1.0)
    C = jnp.concatenate([lo, hi[:, :G - HP]], axis=1)
    Cn = jnp.where(mask, C / sden, 0.0)
    c = Cn @ wt_ref[...] + jnp.where(mask, 1.0, 0.0) * bt_ref[...]
    x = _elu(c)
    h = h_ref[...]
    gi = x @ wih_ref[...] + bih_ref[...]
    gh = h @ whh_ref[...] + bhh_ref[...]
    r = jax.nn.sigmoid(gi[:, :G] + gh[:, :G])
    z = jax.nn.sigmoid(gi[:, G:2 * G] + gh[:, G:2 * G])
    nn_ = jnp.tanh(gi[:, 2 * G:] + r * gh[:, 2 * G:])
    hn = jnp.maximum((1.0 - z) * nn_ + z * h, 0.0)
    hlo_ref[...] = hn[:, :HP]
    hhi_ref[...] = jnp.concatenate(
        [hn[:, HP:], jnp.zeros((hn.shape[0], 2 * HP - G), f32)], axis=1)
    u_ref[...] = hn @ wu_ref[...] + bu_ref[...]
    v_ref[...] = hn @ wv_ref[...]


def _post_round(c2, h, wt, bt, wih, whh, bih, bhh, wu, bu, wv):
    blk = 2000
    grid = N // blk
    return pl.pallas_call(
        _post_round_body,
        grid=(grid,),
        in_specs=[
            pl.BlockSpec((NC, blk, HP), lambda i: (0, i, 0)),
            pl.BlockSpec((blk, G), lambda i: (i, 0)),
            pl.BlockSpec((G, G), lambda i: (0, 0)),
            pl.BlockSpec((1, G), lambda i: (0, 0)),
            pl.BlockSpec((G, 3 * G), lambda i: (0, 0)),
            pl.BlockSpec((G, 3 * G), lambda i: (0, 0)),
            pl.BlockSpec((1, 3 * G), lambda i: (0, 0)),
            pl.BlockSpec((1, 3 * G), lambda i: (0, 0)),
            pl.BlockSpec((G, 1), lambda i: (0, 0)),
            pl.BlockSpec((1, 1), lambda i: (0, 0)),
            pl.BlockSpec((G, 1), lambda i: (0, 0)),
        ],
        out_specs=[
            pl.BlockSpec((blk, HP), lambda i: (i, 0)),
            pl.BlockSpec((blk, HP), lambda i: (i, 0)),
            pl.BlockSpec((blk, 1), lambda i: (i, 0)),
            pl.BlockSpec((blk, 1), lambda i: (i, 0)),
        ],
        out_shape=[
            jax.ShapeDtypeStruct((N, HP), f32),
            jax.ShapeDtypeStruct((N, HP), f32),
            jax.ShapeDtypeStruct((N, 1), f32),
            jax.ShapeDtypeStruct((N, 1), f32),
        ],
    )(c2, h, wt, bt, wih, whh, bih, bhh, wu, bu, wv)


def _edge_exp_body(lg_ref, e_ref):
    e_ref[...] = jnp.exp(lg_ref[...])


def _edge_exp(lg):
    e = pl.pallas_call(
        _edge_exp_body,
        out_shape=jax.ShapeDtypeStruct((E // 128, 128), f32),
    )(lg.reshape(E // 128, 128))
    return e.reshape(E)


def _ro_proj_body(h_ref, wp0_ref, bp0_ref, wp1_ref, bp1_ref, wzv0_ref,
                  wzv1_ref,
                  hp0_ref, hp1_ref, zv0_ref, zv1_ref):
    h = h_ref[...]
    hp0_ref[...] = h @ wp0_ref[...] + bp0_ref[...]
    hp1_ref[...] = h @ wp1_ref[...] + bp1_ref[...]
    zv0_ref[...] = h @ wzv0_ref[...]
    zv1_ref[...] = h @ wzv1_ref[...]


_ro_proj_body = _hiprec(_ro_proj_body)


def _ro_proj(h, wp0, bp0, wp1, bp1, wzv0, wzv1):
    blk = 2000
    grid = N // blk
    full = lambda shp: pl.BlockSpec(shp, lambda i: (0, 0))
    rows = lambda w: pl.BlockSpec((blk, w), lambda i: (i, 0))
    return pl.pallas_call(
        _ro_proj_body,
        grid=(grid,),
        in_specs=[rows(G), full((G, G)), full((1, G)), full((G, G)),
                  full((1, G)), full((G, 1)), full((G, 1))],
        out_specs=[rows(G), rows(G), rows(1), rows(1)],
        out_shape=[
            jax.ShapeDtypeStruct((N, G), f32),
            jax.ShapeDtypeStruct((N, G), f32),
            jax.ShapeDtypeStruct((N, 1), f32),
            jax.ShapeDtypeStruct((N, 1), f32),
        ],
    )(h, wp0, bp0, wp1, bp1, wzv0, wzv1)


def _readout_body(h_ref, hp0_ref, hp1_ref, zv0_ref, zv1_ref,
                  gid_ref, tab_ref,
                  wz0t_ref, bz0_ref, wih0_ref, whh0_ref, bih0_ref, bhh0_ref,
                  wz1t_ref, bz1_ref, wih1_ref, whh1_ref, bih1_ref, bhh1_ref,
                  d0g_ref, d0t_ref, d0b_ref, d1_ref, d1b_ref,
                  pr_ref, prb_ref, out_ref):
    PH = lax.Precision.HIGHEST
    bf16 = jnp.bfloat16
    gid = gid_ref[...]
    M = (jnp.broadcast_to(gid, (B, N)) ==
         lax.broadcasted_iota(jnp.int32, (B, N), 0)).astype(f32)
    Mb = M.astype(bf16)

    def msum(x):
        x1 = x.astype(bf16)
        tail = x - x1.astype(f32)
        return (jax.lax.dot(Mb, x1, preferred_element_type=f32)
                + M @ tail)

    g = msum(h_ref[...])

    def bcast(zt):
        dims = (((0,), (0,)), ((), ()))
        z1 = zt.astype(bf16)
        tail = zt - z1.astype(f32)
        return (lax.dot_general(Mb, z1, dims, preferred_element_type=f32)
                + lax.dot_general(M, tail, dims))

    def round_(g, hp, zv, wzt, bz, wih, whh, bih, bhh):
        ztg = jnp.dot(jnp.maximum(g, 0.0), wzt, precision=PH) + bz
        ec = jnp.exp(_lrelu(bcast(ztg) + zv))
        sb = msum(ec)
        Cg = msum(ec * hp)
        mask = sb > 0
        grp_ = jnp.where(mask, Cg / jnp.where(mask, sb, 1.0), 0.0)
        x = _elu(grp_)
        gi = jnp.dot(x, wih, precision=PH) + bih
        gh = jnp.dot(g, whh, precision=PH) + bhh
        r = jax.nn.sigmoid(gi[:, :G] + gh[:, :G])
        z = jax.nn.sigmoid(gi[:, G:2 * G] + gh[:, G:2 * G])
        nn_ = jnp.tanh(gi[:, 2 * G:] + r * gh[:, 2 * G:])
        return jnp.maximum((1.0 - z) * nn_ + z * g, 0.0)

    g = round_(g, hp0_ref[...], zv0_ref[...], wz0t_ref[...], bz0_ref[...],
               wih0_ref[...], whh0_ref[...], bih0_ref[...], bhh0_ref[...])
    g = round_(g, hp1_ref[...], zv1_ref[...], wz1t_ref[...], bz1_ref[...],
               wih1_ref[...], whh1_ref[...], bih1_ref[...], bhh1_ref[...])

    x = jnp.maximum(jnp.dot(g, d0g_ref[...], precision=PH)
                    + jnp.dot(tab_ref[...], d0t_ref[...], precision=PH)
                    + d0b_ref[...], 0.0)
    x = jnp.maximum(jnp.dot(x, d1_ref[...], precision=PH) + d1b_ref[...], 0.0)
    out_ref[...] = jnp.dot(x, pr_ref[...], precision=PH) + prb_ref[...]


def _readout(h, hp0, hp1, zv0, zv1, gid2d, tab, *ws):
    return pl.pallas_call(
        _readout_body,
        out_shape=jax.ShapeDtypeStruct((B, 1), f32),
        compiler_params=pltpu.CompilerParams(
            vmem_limit_bytes=100 * 1024 * 1024),
    )(h, hp0, hp1, zv0, zv1, gid2d, tab, *ws)


@functools.cache
def _mesh():
    return plsc.VectorSubcoreMesh(
        core_axis_name="c", subcore_axis_name="s",
        num_cores=NC, num_subcores=NS)


_SC_PARAMS = pltpu.CompilerParams(
    use_tc_tiling_on_sc=False, needs_layout_passes=False)


def _zero_shared(Cs, buf_v, sid):
    z16 = jnp.zeros((16,), f32)

    def zrow(r, _):
        def zcol(j, _):
            buf_v[r, pl.ds(j * 16, 16)] = z16
            return 0
        return lax.fori_loop(0, NH, zcol, 0)

    lax.fori_loop(0, RZ, zrow, 0)

    def zcopy(k, _):
        pltpu.sync_copy(buf_v, Cs.at[pl.ds(sid * RPT + k * RZ, RZ)])
        return 0

    lax.fori_loop(0, RPT // RZ, zcopy, 0)

    @pl.when(sid == NS - 1)
    def _():
        pltpu.sync_copy(buf_v.at[pl.ds(0, NTAIL)],
                        Cs.at[pl.ds(NS * RPT, NTAIL)])


def _write_shared(Cs, buf_v, out_hbm, cid, sid):
    def wcopy(k, _):
        row0 = sid * RPT + k * RZ
        pltpu.sync_copy(Cs.at[pl.ds(row0, RZ)], buf_v)
        pltpu.sync_copy(buf_v, out_hbm.at[cid, pl.ds(row0, RZ)])
        return 0

    lax.fori_loop(0, RPT // RZ, wcopy, 0)

    @pl.when(sid == NS - 1)
    def _():
        pltpu.sync_copy(Cs.at[pl.ds(NS * RPT, NTAIL)],
                        buf_v.at[pl.ds(0, NTAIL)])
        pltpu.sync_copy(buf_v.at[pl.ds(0, NTAIL)],
                        out_hbm.at[cid, pl.ds(NS * RPT, NTAIL)])


def _bf16r(x):
    b = plsc.bitcast(x, jnp.uint32)
    lsb = (b >> 16) & jnp.uint32(1)
    b = (b + jnp.uint32(0x7FFF) + lsb) & jnp.uint32(0xFFFF0000)
    return plsc.bitcast(b, f32)


def _ebcast(e_v, i):
    return plsc.load_gather(e_v, [jnp.full((16,), i, jnp.int32)])


def _sc_ctx_logits_body(src_hbm, dst_hbm, pn_hbm, pelo_hbm, pehi_hbm, u_hbm,
                        w2b_hbm, lg_hbm,
                        u_v, w2b_v, dotp_v, lg_v,
                        s_0, d_0, s2_0, rlo_0, rhi_0, plo_0, phi_0,
                        s_1, d_1, s2_1, rlo_1, rhi_1, plo_1, phi_1,
                        semi0, semg0, semi1, semg1):
    cid = lax.axis_index("c")
    sid = lax.axis_index("s")
    wid = cid * NS + sid
    pltpu.sync_copy(u_hbm, u_v)
    pltpu.sync_copy(w2b_hbm, w2b_v)

    base0 = wid * EPW
    bufs = ((s_0, d_0, s2_0, rlo_0, rhi_0, plo_0, phi_0, semi0, semg0),
            (s_1, d_1, s2_1, rlo_1, rhi_1, plo_1, phi_1, semi1, semg1))

    def idx_descs(slot, t):
        sv, dv = bufs[slot][0], bufs[slot][1]
        semi = bufs[slot][7]
        base = base0 + t * KC
        return (pltpu.make_async_copy(src_hbm.at[pl.ds(base, KC)], sv, semi),
                pltpu.make_async_copy(dst_hbm.at[pl.ds(base, KC)], dv, semi))

    def row_descs(slot, t):
        sv, _, s2, rlo, rhi, plo, phi, _, semg = bufs[slot]
        base = base0 + t * KC
        return (
            pltpu.make_async_copy(pn_hbm.at[sv], rlo, semg),
            pltpu.make_async_copy(pn_hbm.at[s2], rhi, semg),
            pltpu.make_async_copy(pelo_hbm.at[pl.ds(base, KC)], plo, semg),
            pltpu.make_async_copy(pehi_hbm.at[pl.ds(base, KC)], phi, semg),
        )

    def prep_rows(slot, t):
        sv, _, s2 = bufs[slot][0], bufs[slot][1], bufs[slot][2]

        def g0(gi, _):
            s2[pl.ds(gi * 16, 16)] = sv[pl.ds(gi * 16, 16)] + N
            return 0

        lax.fori_loop(0, NGRP, g0, 0)
        for dsc in row_descs(slot, t):
            dsc.start()

    def compute(slot, t):
        _, dv, _, rlo, rhi, plo, phi, _, _ = bufs[slot]
        for dsc in row_descs(slot, t):
            dsc.wait()

        def edge(i, _):
            def flo(j, acc):
                q = rlo[i, pl.ds(j * 16, 16)] + plo[i, pl.ds(j * 16, 16)]
                he = jnp.where(q >= 0, q, 0.01 * q)
                return acc + he * w2b_v[pl.ds(j * 16, 16)]

            def fhi(j, acc):
                q = rhi[i, pl.ds(j * 16, 16)] + phi[i, pl.ds(j * 16, 16)]
                he = jnp.where(q >= 0, q, 0.01 * q)
                return acc + he * w2b_v[pl.ds((NH + j) * 16, 16)]

            acc = lax.fori_loop(0, NH, flo, jnp.zeros((16,), f32))
            acc = lax.fori_loop(0, NH - 1, fhi, acc)
            dotp_v[i, pl.ds(0, 16)] = acc
            return 0

        lax.fori_loop(0, KC, edge, 0)

        def grp(gi, _):
            eids = lax.iota(jnp.int32, 16) + gi * 16
            dstv = dv[pl.ds(gi * 16, 16)]
            uv = plsc.load_gather(u_v, [dstv])

            def dj(j, acc):
                jv = jnp.full((16,), j, jnp.int32)
                return acc + plsc.load_gather(dotp_v, [eids, jv])

            acc = lax.fori_loop(0, 16, dj, jnp.zeros((16,), f32))
            lg = uv + acc
            lg_v[pl.ds(gi * 16, 16)] = jnp.where(lg >= 0, lg, 0.01 * lg)
            return 0

        lax.fori_loop(0, NGRP, grp, 0)
        pltpu.sync_copy(lg_v, lg_hbm.at[pl.ds(base0 + t * KC, KC)])

    for dsc in idx_descs(0, 0):
        dsc.start()
    for dsc in idx_descs(0, 0):
        dsc.wait()
    prep_rows(0, 0)
    for dsc in idx_descs(1, 1):
        dsc.start()

    def pair(tp, _):
        for b in (0, 1):
            t = 2 * tp + b
            slot, nslot = b, 1 - b
            nt = t + 1

            @pl.when(nt < NCHUNKW)
            def _():
                for dsc in idx_descs(nslot, nt):
                    dsc.wait()
                prep_rows(nslot, nt)

            compute(slot, t)

            @pl.when(t + 2 < NCHUNKW)
            def _():
                for dsc in idx_descs(slot, t + 2):
                    dsc.start()
        return 0

    lax.fori_loop(0, NCHUNKW // 2, pair, 0)
    if NCHUNKW % 2:
        compute(0, NCHUNKW - 1)


def _sc_lay_logits_body(src_hbm, dst_hbm, u_hbm, v_hbm, lg_hbm,
                        u_v, v_v, src_v, dst_v, lg_v):
    cid = lax.axis_index("c")
    sid = lax.axis_index("s")
    wid = cid * NS + sid
    pltpu.sync_copy(u_hbm, u_v)
    pltpu.sync_copy(v_hbm, v_v)

    base0 = wid * EPW

    def chunk(t, _):
        base = base0 + t * KC
        pltpu.sync_copy(src_hbm.at[pl.ds(base, KC)], src_v)
        pltpu.sync_copy(dst_hbm.at[pl.ds(base, KC)], dst_v)

        def grp(gi, _):
            srcv = src_v[pl.ds(gi * 16, 16)]
            dstv = dst_v[pl.ds(gi * 16, 16)]
            uv = plsc.load_gather(u_v, [dstv])
            vv = plsc.load_gather(v_v, [srcv])
            lg = uv + vv
            lg_v[pl.ds(gi * 16, 16)] = jnp.where(lg >= 0, lg, 0.01 * lg)
            return 0

        lax.fori_loop(0, NGRP, grp, 0)
        pltpu.sync_copy(lg_v, lg_hbm.at[pl.ds(base, KC)])
        return 0

    lax.fori_loop(0, NCHUNKW, chunk, 0)


def _scale_half_rows(rows, ev, cid, extra=None):
    z16 = jnp.zeros((16,), f32)
    lane8 = lax.iota(jnp.int32, 16) == 8

    def scale(i, _):
        esv = _ebcast(ev, i)
        eblend = jnp.where(jnp.logical_and(lane8, cid == 1), esv, z16)

        def chunk_val(j):
            r = rows[i, pl.ds(j * 16, 16)]
            if extra is not None:
                q = r + extra[i, pl.ds(j * 16, 16)]
                r = jnp.where(q >= 0, q, 0.01 * q)
            return r

        def sj(j, _):
            rows[i, pl.ds(j * 16, 16)] = chunk_val(j) * esv
            return 0

        lax.fori_loop(0, NH - 2, sj, 0)
        j = NH - 2
        rows[i, pl.ds(j * 16, 16)] = chunk_val(j) * esv + eblend
        j = NH - 1
        rows[i, pl.ds(j * 16, 16)] = chunk_val(j) * esv
        return 0

    lax.fori_loop(0, KC, scale, 0)


def _make_scatter_body(with_pe):
    def body(src_hbm, dst_hbm, tab_hbm, pelo_hbm, pehi_hbm, e_hbm, out_hbm,
             Cs,
             s_0, d_0, s2_0, e_0, r_0, p_0,
             s_1, d_1, s2_1, e_1, r_1, p_1,
             buf_v, semi0, semg0, semi1, semg1):
        cid = lax.axis_index("c")
        sid = lax.axis_index("s")
        _zero_shared(Cs, buf_v, sid)
        plsc.subcore_barrier()

        base0 = sid * EPT
        rowoff = cid * N
        bufs = ((s_0, d_0, s2_0, e_0, r_0, p_0, semi0, semg0),
                (s_1, d_1, s2_1, e_1, r_1, p_1, semi1, semg1))

        def idx_descs(slot, t):
            sv, dv, _, ev = bufs[slot][:4]
            semi = bufs[slot][6]
            base = base0 + t * KC
            ds = [pltpu.make_async_copy(
                      src_hbm.at[pl.ds(base, KC)], sv, semi),
                  pltpu.make_async_copy(
                      dst_hbm.at[pl.ds(base, KC)], dv, semi),
                  pltpu.make_async_copy(
                      e_hbm.at[pl.ds(base, KC)], ev, semi)]
            return ds

        def pe_descs(slot, t, c):
            pv = bufs[slot][5]
            semi = bufs[slot][6]
            base = base0 + t * KC
            hbm = pelo_hbm if c == 0 else pehi_hbm
            return pltpu.make_async_copy(hbm.at[pl.ds(base, KC)], pv, semi)

        def row_desc(slot):
            s2, rv = bufs[slot][2], bufs[slot][4]
            semg = bufs[slot][7]
            return pltpu.make_async_copy(tab_hbm.at[s2], rv, semg)

        def prep_rows(slot, t, cid):
            sv, s2 = bufs[slot][0], bufs[slot][2]

            def g0(gi, _):
                s2[pl.ds(gi * 16, 16)] = sv[pl.ds(gi * 16, 16)] + rowoff
                return 0

            lax.fori_loop(0, NGRP, g0, 0)
            row_desc(slot).start()
            if with_pe:
                @pl.when(cid == 0)
                def _():
                    pe_descs(slot, t, 0).start()

                @pl.when(cid == 1)
                def _():
                    pe_descs(slot, t, 1).start()

        def wait_rows(slot, t, cid):
            row_desc(slot).wait()
            if with_pe:
                @pl.when(cid == 0)
                def _():
                    pe_descs(slot, t, 0).wait()

                @pl.when(cid == 1)
                def _():
                    pe_descs(slot, t, 1).wait()

        def scale_scatter(slot, t):
            dv, ev, rv, pv = (bufs[slot][1], bufs[slot][3], bufs[slot][4],
                              bufs[slot][5])
            wait_rows(slot, t, cid)
            _scale_half_rows(rv, ev, cid, extra=pv if with_pe else None)
            pltpu.sync_copy(rv, Cs.at[dv], add=True)

        for dsc in idx_descs(0, 0):
            dsc.start()
        for dsc in idx_descs(0, 0):
            dsc.wait()
        prep_rows(0, 0, cid)
        for dsc in idx_descs(1, 1):
            dsc.start()

        def pair(tp, _):
            for b in (0, 1):
                t = 2 * tp + b
                slot, nslot = b, 1 - b
                nt = t + 1

                @pl.when(nt < NCHUNK)
                def _():
                    for dsc in idx_descs(nslot, nt):
                        dsc.wait()
                    prep_rows(nslot, nt, cid)

                scale_scatter(slot, t)

                @pl.when(t + 2 < NCHUNK)
                def _():
                    for dsc in idx_descs(slot, t + 2):
                        dsc.start()
            return 0

        lax.fori_loop(0, NCHUNK // 2, pair, 0)
        if NCHUNK % 2:
            scale_scatter(0, NCHUNK - 1)
        plsc.subcore_barrier()
        _write_shared(Cs, buf_v, out_hbm, cid, sid)

    return body


_sc_scatter_body = _make_scatter_body(with_pe=False)
_sc_ctx_scatter_body = _make_scatter_body(with_pe=True)


def _scatter_scratch():
    one_slot = [
        pltpu.VMEM((KC,), jnp.int32),
        pltpu.VMEM((KC,), jnp.int32),
        pltpu.VMEM((KC,), jnp.int32),
        pltpu.VMEM((KC,), f32),
        pltpu.VMEM((KC, HP), f32),
        pltpu.VMEM((KC, HP), f32),
    ]
    return ([pltpu.VMEM_SHARED((N, HP), f32)] + one_slot + one_slot +
            [pltpu.VMEM((RZ, HP), f32),
             pltpu.SemaphoreType.DMA, pltpu.SemaphoreType.DMA,
             pltpu.SemaphoreType.DMA, pltpu.SemaphoreType.DMA])


@functools.cache
def _sc_ctx_logits_kernel():
    one_slot = [
        pltpu.VMEM((KC,), jnp.int32),
        pltpu.VMEM((KC,), jnp.int32),
        pltpu.VMEM((KC,), jnp.int32),
        pltpu.VMEM((KC, HP), f32),
        pltpu.VMEM((KC, HP), f32),
        pltpu.VMEM((KC, HP), f32),
        pltpu.VMEM((KC, HP), f32),
    ]
    return pl.kernel(
        _sc_ctx_logits_body,
        out_type=jax.ShapeDtypeStruct((E,), f32),
        mesh=_mesh(),
        compiler_params=_SC_PARAMS,
        scratch_types=(
            [pltpu.VMEM((N,), f32), pltpu.VMEM((GP,), f32),
             pltpu.VMEM((KC, 16), f32), pltpu.VMEM((KC,), f32)] +
            one_slot + one_slot +
            [pltpu.SemaphoreType.DMA, pltpu.SemaphoreType.DMA,
             pltpu.SemaphoreType.DMA, pltpu.SemaphoreType.DMA]),
    )


@functools.cache
def _sc_lay_logits_kernel():
    return pl.kernel(
        _sc_lay_logits_body,
        out_type=jax.ShapeDtypeStruct((E,), f32),
        mesh=_mesh(),
        compiler_params=_SC_PARAMS,
        scratch_types=[
            pltpu.VMEM((N,), f32),
            pltpu.VMEM((N,), f32),
            pltpu.VMEM((KC,), jnp.int32),
            pltpu.VMEM((KC,), jnp.int32),
            pltpu.VMEM((KC,), f32),
        ],
    )


@functools.cache
def _sc_scatter_kernel():
    return pl.kernel(
        _sc_scatter_body,
        out_type=jax.ShapeDtypeStruct((NC, N, HP), f32),
        mesh=_mesh(),
        compiler_params=_SC_PARAMS,
        scratch_types=_scatter_scratch(),
    )


@functools.cache
def _sc_ctx_scatter_kernel():
    return pl.kernel(
        _sc_ctx_scatter_body,
        out_type=jax.ShapeDtypeStruct((NC, N, HP), f32),
        mesh=_mesh(),
        compiler_params=_SC_PARAMS,
        scratch_types=_scatter_scratch(),
    )


def _sc_ctx(srcv, dstv, pn_stk, pe_lo, pe_hi, u0, w2b):
    lg = _sc_ctx_logits_kernel()(srcv, dstv, pn_stk, pe_lo, pe_hi, u0, w2b)
    e = _edge_exp(lg)
    return _sc_ctx_scatter_kernel()(srcv, dstv, pn_stk, pe_lo, pe_hi, e)


def _sc_layer(srcv, dstv, h_stk, u, v):
    lg = _sc_lay_logits_kernel()(srcv, dstv, u, v)
    e = _edge_exp(lg)
    return _sc_scatter_kernel()(srcv, dstv, h_stk, h_stk, h_stk, e)


def kernel(node_feats, edge_feats, tab_feats, edge_index, graph_ids,
           gc_Wpn, gc_bpn, gc_Wpe1, gc_bpe1, gc_Wpe2, gc_bpe2, gc_Wet, gc_bet,
           gc_Wih, gc_Whh, gc_bih, gc_bhh,
           gl_Wpe, gl_bpe, gl_Wpn, gl_bpn, gl_Wih, gl_Whh, gl_bih, gl_bhh,
           ro_Wz, ro_bz, ro_Wp, ro_bp, ro_Wih, ro_Whh, ro_bih, ro_bhh,
           d0_W, d0_b, d1_W, d1_b, pr_W, pr_b):
    src = edge_index[0]
    dst = edge_index[1]

    wpe1n = jnp.pad(gc_Wpe1[:DN], ((0, 0), (0, GP - G)))
    wpe1e = jnp.pad(gc_Wpe1[DN:], ((0, 0), (0, GP - G)))
    bpe1p = jnp.pad(gc_bpe1, (0, GP - G)).reshape(1, GP)
    w2u = gc_Wpe2[:G]
    w2b = jnp.pad(gc_Wpe2[G:, 0], (0, GP - G))
    bpe2 = gc_bpe2.reshape(1, 1)
    row = lambda b: b.reshape(1, -1)

    hv, pn, u0 = _prep_nodes(node_feats, gc_Wpn, row(gc_bpn), wpe1n, w2u, bpe2)
    pe = _edge_proj(edge_feats, wpe1e, bpe1p)
    pad16 = lambda a: jnp.pad(a, ((0, 0), (0, 2 * HP - GP)))
    pn_stk = jnp.concatenate([pn[:, :HP], pad16(pn[:, HP:])], axis=0)
    pe_lo = pe[:, :HP]
    pe_hi = pad16(pe[:, HP:])

    c2 = _sc_ctx(src, dst, pn_stk, pe_lo, pe_hi, u0.reshape(N), w2b)
    hlo, hhi, u, v = _post_round(
        c2, hv, gc_Wet, row(gc_bet), gc_Wih, gc_Whh, row(gc_bih), row(gc_bhh),
        gl_Wpe[0][:G], gl_bpe[0].reshape(1, 1), gl_Wpe[0][G:])

    c2 = _sc_layer(src, dst, jnp.concatenate([hlo, hhi], axis=0),
                   u.reshape(N), v.reshape(N))
    h = jnp.concatenate([hlo, hhi[:, :G - HP]], axis=1)
    hlo, hhi, u, v = _post_round(
        c2, h, gl_Wpn[0], row(gl_bpn[0]), gl_Wih[0], gl_Whh[0],
        row(gl_bih[0]), row(gl_bhh[0]),
        gl_Wpe[1][:G], gl_bpe[1].reshape(1, 1), gl_Wpe[1][G:])

    c2 = _sc_layer(src, dst, jnp.concatenate([hlo, hhi], axis=0),
                   u.reshape(N), v.reshape(N))
    h = jnp.concatenate([hlo, hhi[:, :G - HP]], axis=1)
    hlo, hhi, _, _ = _post_round(
        c2, h, gl_Wpn[1], row(gl_bpn[1]), gl_Wih[1], gl_Whh[1],
        row(gl_bih[1]), row(gl_bhh[1]),
        jnp.zeros((G, 1), f32), jnp.zeros((1, 1), f32), jnp.zeros((G, 1), f32))
    h = jnp.concatenate([hlo, hhi[:, :G - HP]], axis=1)

    hp0, hp1, zv0, zv1 = _ro_proj(
        h, ro_Wp[0], row(ro_bp[0]), ro_Wp[1], row(ro_bp[1]),
        ro_Wz[0][G:], ro_Wz[1][G:])
    ws = (ro_Wz[0][:G], ro_bz[0].reshape(1, 1),
          ro_Wih[0], ro_Whh[0], row(ro_bih[0]), row(ro_bhh[0]),
          ro_Wz[1][:G], ro_bz[1].reshape(1, 1),
          ro_Wih[1], ro_Whh[1], row(ro_bih[1]), row(ro_bhh[1]),
          d0_W[:G], d0_W[G:], row(d0_b), d1_W, row(d1_b), pr_W,
          pr_b.reshape(1, 1))
    return _readout(h, hp0, hp1, zv0, zv1,
                    graph_ids.reshape(1, N), tab_feats, *ws)
